# baseline (speedup 1.0000x reference)
import dataclasses
import functools

import jax
import jax.numpy as jnp
from jax import lax
from jax.experimental import pallas as pl
from jax.experimental.pallas import tpu as pltpu
from jax.experimental.pallas import tpu_sc as plsc

BATCH = 16384
NUM_AREA = 100000
EMBED_DIM = 64
ROW = 2 * EMBED_DIM
NC = 2
NS = 16
NW = NC * NS
B_PER_W = BATCH // NW
CHUNK = 128
NCH = B_PER_W // CHUNK
LANES = 16

_MESH = plsc.VectorSubcoreMesh(core_axis_name="c", subcore_axis_name="s")
_CP = pltpu.CompilerParams()
if "needs_layout_passes" in pltpu.CompilerParams.__dataclass_fields__:
    _CP = dataclasses.replace(_CP, needs_layout_passes=False)

_IDX = pltpu.VMEM((B_PER_W,), jnp.int32)
_BIG = pltpu.VMEM((CHUNK, ROW), jnp.float32)
_DMA = pltpu.SemaphoreType.DMA


@functools.partial(
    pl.kernel, mesh=_MESH, compiler_params=_CP, out_type=(),
    scratch_types=[_IDX, _IDX, _BIG, _BIG, _DMA, _DMA, _DMA, _DMA, _DMA],
)
def _even_kernel(g_hbm, a_hbm, wga_hbm, out_ref,
                 gav, dv, tmp0, tmp1, semi, semg0, semg1, sems0, sems1):
    tmps = (tmp0, tmp1)
    semg = (semg0, semg1)
    sems = (sems0, sems1)
    wid = lax.axis_index("s") * NC + lax.axis_index("c")
    base = wid * B_PER_W
    iot = lax.iota(jnp.int32, LANES)
    loads = [pltpu.async_copy(src.at[pl.ds(base, B_PER_W)], dst, semi)
             for src, dst in ((g_hbm, gav), (a_hbm, dv))]
    for h in loads:
        h.wait()
    for t in range(B_PER_W // LANES):
        s = pl.ds(t * LANES, LANES)
        gav.at[s][...] = gav.at[s][...] * 7 + dv.at[s][...]
        dv.at[s][...] = iot * 2 + (2 * base + 2 * t * LANES)

    def gather(c):
        s = c % 2
        off = pl.ds(c * CHUNK, CHUNK)
        return pltpu.async_copy(wga_hbm.at[gav.at[off]], tmps[s], semg[s])

    pend_g = {0: gather(0), 1: None}
    pend_s = {0: None, 1: None}
    for c in range(NCH):
        s = c % 2
        pend_g[s].wait()
        if c + 1 < NCH:
            if pend_s[1 - s] is not None:
                pend_s[1 - s].wait()
                pend_s[1 - s] = None
            pend_g[1 - s] = gather(c + 1)
        pend_s[s] = pltpu.async_copy(
            tmps[s], out_ref.at[dv.at[pl.ds(c * CHUNK, CHUNK)]], sems[s])
    for s in (0, 1):
        if pend_s[s] is not None:
            pend_s[s].wait()


@functools.partial(
    pl.kernel, mesh=_MESH, compiler_params=_CP, out_type=(),
    scratch_types=[_IDX, _IDX, _IDX, _IDX, _BIG, _BIG, _BIG, _BIG,
                   _DMA, _DMA, _DMA, _DMA, _DMA],
)
def _odd_kernel(o_hbm, ar_hbm, wocc_hbm, xar_hbm, out_ref,
                ov, arv, hv, dv, tmp0, tmp1, area0, area1,
                semi, semg0, semg1, sems0, sems1):
    tmps = (tmp0, tmp1)
    areas = (area0, area1)
    semg = (semg0, semg1)
    sems = (sems0, sems1)
    wid = lax.axis_index("s") * NC + lax.axis_index("c")
    base = wid * B_PER_W
    iot = lax.iota(jnp.int32, LANES)
    loads = [pltpu.async_copy(src.at[pl.ds(base, B_PER_W)], dst, semi)
             for src, dst in ((o_hbm, ov), (ar_hbm, arv))]
    for h in loads:
        h.wait()
    for t in range(B_PER_W // LANES):
        s = pl.ds(t * LANES, LANES)
        ar16 = arv.at[s][...]
        hv.at[s][...] = lax.shift_left(jnp.bitwise_and(ar16, 1), 6)
        arv.at[s][...] = lax.shift_right_logical(ar16, 1)
        dv.at[s][...] = iot * 2 + (2 * base + 2 * t * LANES + 1)

    def gathers(c):
        s = c % 2
        off = pl.ds(c * CHUNK, CHUNK)
        return [
            pltpu.async_copy(wocc_hbm.at[ov.at[off]], tmps[s], semg[s]),
            pltpu.async_copy(xar_hbm.at[arv.at[off]], areas[s], semg[s]),
        ]

    def select_half(c):
        s = c % 2
        @pl.loop(0, CHUNK, step=LANES)
        def _(j0):
            rowv = iot + j0
            hvv = hv.at[pl.ds(c * CHUNK + j0, LANES)][...]
            colv = iot * 0 + EMBED_DIM
            for cc in range(EMBED_DIM):
                vals = plsc.load_gather(areas[s], [rowv, hvv + cc])
                plsc.store_scatter(tmps[s], [rowv, colv + cc], vals)

    pend_g = {0: gathers(0), 1: None}
    pend_s = {0: None, 1: None}
    for c in range(NCH):
        s = c % 2
        for h in pend_g[s]:
            h.wait()
        if c + 1 < NCH:
            if pend_s[1 - s] is not None:
                pend_s[1 - s].wait()
                pend_s[1 - s] = None
            pend_g[1 - s] = gathers(c + 1)
        select_half(c)
        pend_s[s] = pltpu.async_copy(
            tmps[s], out_ref.at[dv.at[pl.ds(c * CHUNK, CHUNK)]], sems[s])
    for s in (0, 1):
        if pend_s[s] is not None:
            pend_s[s].wait()


def kernel(gender_idx, age_idx, occupation_idx, area_idx, u_id,
           W_gender, W_age, W_occupation, W_area):
    del u_id
    g = gender_idx.astype(jnp.int32)
    a = age_idx.astype(jnp.int32)
    o = occupation_idx.astype(jnp.int32)
    ar = area_idx.astype(jnp.int32)

    W_ga = jnp.concatenate(
        [jnp.repeat(W_gender, 7, axis=0), jnp.tile(W_age, (2, 1))], axis=1)
    W_occ128 = jnp.pad(W_occupation, ((0, 0), (0, EMBED_DIM)))
    X_area = W_area.reshape(-1, ROW)

    out_ref = jax.new_ref(jnp.zeros((2 * BATCH, ROW), jnp.float32))
    _even_kernel(g, a, W_ga, out_ref)
    _odd_kernel(o, ar, W_occ128, X_area, out_ref)
    return out_ref[...].reshape(BATCH, 4 * EMBED_DIM)
